# baseline (speedup 1.0000x reference)
import functools
import math

import jax
import jax.numpy as jnp
from jax import lax
from jax.experimental import pallas as pl
from jax.experimental.pallas import tpu as pltpu
from jax.experimental.pallas import tpu_sc as plsc

_SCALE = 64.0
_MARGIN = 0.5
_COS_M = math.cos(_MARGIN)
_SIN_M = math.sin(_MARGIN)

_NC = 2
_NS = 16
_NW = _NC * _NS
_LANES = 16

_N = 1024
_C = 100000
_PER_W_ROWS = _N // _NW
_PER_W = _PER_W_ROWS * _C
_CHUNK = 20000
_NCHUNK = _PER_W // _CHUNK
_VECS = _CHUNK // _LANES


def _fix_from_x(x):
    a = jnp.maximum(1.0 - x * x, 1e-12)
    bits = lax.bitcast_convert_type(a, jnp.int32)
    y = lax.bitcast_convert_type((bits >> 1) + 0x1FBD1DF5, jnp.float32)
    for _ in range(3):
        y = 0.5 * (y + a / y)
    return _SCALE * (x * _COS_M - y * _SIN_M)


def _sc_arc_margin(flat_cos, label):
    mesh = plsc.VectorSubcoreMesh(core_axis_name="c", subcore_axis_name="s")

    @functools.partial(
        pl.kernel,
        mesh=mesh,
        out_type=jax.ShapeDtypeStruct((_N * _C,), jnp.float32),
        scratch_types=[
            pltpu.VMEM((_PER_W_ROWS,), jnp.int32),
            pltpu.VMEM((_PER_W_ROWS,), jnp.float32),
            pltpu.VMEM((_CHUNK,), jnp.float32),
            pltpu.VMEM((_CHUNK,), jnp.float32),
            pltpu.VMEM((_CHUNK,), jnp.float32),
            pltpu.VMEM((_CHUNK,), jnp.float32),
            pltpu.SemaphoreType.DMA,
            pltpu.SemaphoreType.DMA,
            pltpu.SemaphoreType.DMA,
            pltpu.SemaphoreType.DMA,
            pltpu.SemaphoreType.DMA,
        ],
    )
    def k(flat_hbm, lbl_hbm, out_hbm, idx_v, val_v,
          in0, in1, out0, out1, sg, si0, si1, so0, so1):
        wid = lax.axis_index("s") * _NC + lax.axis_index("c")
        base_row = wid * _PER_W_ROWS
        base = wid * _PER_W

        pltpu.sync_copy(lbl_hbm.at[pl.ds(base_row, _PER_W_ROWS)], idx_v)
        for j in range(_PER_W_ROWS // _LANES):
            lbl = idx_v[pl.ds(j * _LANES, _LANES)]
            row = lax.iota(jnp.int32, _LANES) + (base_row + j * _LANES)
            idx_v[pl.ds(j * _LANES, _LANES)] = row * _C + jnp.maximum(lbl, 0)
        pltpu.async_copy(flat_hbm.at[idx_v], val_v, sg).wait()
        for j in range(_PER_W_ROWS // _LANES):
            x = val_v[pl.ds(j * _LANES, _LANES)]
            val_v[pl.ds(j * _LANES, _LANES)] = _fix_from_x(x)

        ins = (in0, in1)
        outs = (out0, out1)
        sins = (si0, si1)
        souts = (so0, so1)

        def load(c, b):
            pltpu.async_copy(flat_hbm.at[pl.ds(base + c * _CHUNK, _CHUNK)],
                             ins[b], sins[b])

        def store(c, b):
            pltpu.async_copy(outs[b], out_hbm.at[pl.ds(base + c * _CHUNK, _CHUNK)],
                             souts[b])

        def wait_load(c, b):
            pltpu.make_async_copy(flat_hbm.at[pl.ds(base + c * _CHUNK, _CHUNK)],
                                  ins[b], sins[b]).wait()

        def wait_store(c, b):
            pltpu.make_async_copy(outs[b], out_hbm.at[pl.ds(base + c * _CHUNK, _CHUNK)],
                                  souts[b]).wait()

        load(0, 0)
        load(1, 1)

        @pl.loop(0, _NCHUNK // 2)
        def _pair(g):
            for b in range(2):
                c = 2 * g + b
                wait_load(c, b)

                @pl.when(g >= 1)
                def _():
                    wait_store(c - 2, b)

                src = ins[b]
                dst = outs[b]

                @pl.loop(0, _VECS, unroll=10)
                def _scale(i):
                    dst[pl.ds(i * _LANES, _LANES)] = (
                        src[pl.ds(i * _LANES, _LANES)] * _SCALE)

                store(c, b)

                @pl.when(g <= _NCHUNK // 2 - 2)
                def _():
                    load(c + 2, b)

        wait_store(_NCHUNK - 2, 0)
        wait_store(_NCHUNK - 1, 1)

        pltpu.async_copy(val_v, out_hbm.at[idx_v], sg).wait()

    return k(flat_cos, label)


def kernel(cosine, label):
    n, c = cosine.shape
    lbl = label.astype(jnp.int32)
    out = _sc_arc_margin(cosine.reshape(-1), lbl)
    return out.reshape(n, c)


# speedup vs baseline: 4.7478x; 4.7478x over previous
import functools
import math

import jax
import jax.numpy as jnp
from jax import lax
from jax.experimental import pallas as pl
from jax.experimental.pallas import tpu as pltpu
from jax.experimental.pallas import tpu_sc as plsc

_SCALE = 64.0
_INV_SCALE = 1.0 / 64.0
_MARGIN = 0.5
_COS_M = math.cos(_MARGIN)
_SIN_M = math.sin(_MARGIN)

_NC = 2
_NS = 16
_NW = _NC * _NS
_LANES = 16

_N = 1024
_C = 100000
_CR = 32
_NCHUNK = _C // _CR
_TSTEPS = 100
_NGRP = _N // _LANES


def _fix_from_x(x):
    a = jnp.maximum(1.0 - x * x, 1e-12)
    bits = lax.bitcast_convert_type(a, jnp.int32)
    y = lax.bitcast_convert_type((bits >> 1) + 0x1FBD1DF5, jnp.float32)
    for _ in range(3):
        y = 0.5 * (y + a / y)
    return _SCALE * (x * _COS_M - y * _SIN_M)


def _sc_arc_margin_t(xt, label):
    mesh = plsc.VectorSubcoreMesh(core_axis_name="c", subcore_axis_name="s")

    @functools.partial(
        pl.kernel,
        mesh=mesh,
        compiler_params=pltpu.CompilerParams(needs_layout_passes=False),
        out_type=jax.ShapeDtypeStruct((_C, _N), jnp.float32),
        scratch_types=[
            pltpu.VMEM((_N,), jnp.int32),
            pltpu.VMEM((_CR, _N), jnp.float32),
            pltpu.VMEM((_CR, _N), jnp.float32),
            pltpu.SemaphoreType.DMA,
            pltpu.SemaphoreType.DMA,
            pltpu.SemaphoreType.DMA,
            pltpu.SemaphoreType.DMA,
        ],
    )
    def k(x_hbm, lbl_hbm, out_hbm, lbl_v, buf0, buf1, si0, si1, so0, so1):
        wid = lax.axis_index("s") * _NC + lax.axis_index("c")

        pltpu.sync_copy(lbl_hbm, lbl_v)

        bufs = (buf0, buf1)
        sins = (si0, si1)
        souts = (so0, so1)

        def load(c, b):
            pltpu.async_copy(x_hbm.at[pl.ds(c * _CR, _CR), :], bufs[b], sins[b])

        def wait_load(c, b):
            pltpu.make_async_copy(x_hbm.at[pl.ds(c * _CR, _CR), :],
                                  bufs[b], sins[b]).wait()

        def store(c, b):
            pltpu.async_copy(bufs[b], out_hbm.at[pl.ds(c * _CR, _CR), :],
                             souts[b])

        def wait_store(c, b):
            pltpu.make_async_copy(bufs[b], out_hbm.at[pl.ds(c * _CR, _CR), :],
                                  souts[b]).wait()

        def process(c, b):
            buf = bufs[b]
            r0 = c * _CR

            @plsc.parallel_loop(0, _CR)
            def _row(r):
                for i in range(_N // _LANES):
                    buf[r, pl.ds(i * _LANES, _LANES)] = (
                        buf[r, pl.ds(i * _LANES, _LANES)] * _SCALE)

            @pl.loop(0, _NGRP)
            def _grp(jv):
                lbl = lbl_v[pl.ds(jv * _LANES, _LANES)]
                off = lbl - r0
                m = (off >= 0) & (off < _CR)
                hits = plsc.all_reduce_population_count(m)

                @pl.when(jnp.max(hits) > 0)
                def _():
                    colv = lax.iota(jnp.int32, _LANES) + jv * _LANES
                    offr = jnp.minimum(jnp.maximum(off, 0), _CR - 1)
                    y = plsc.load_gather(buf, [offr, colv], mask=m)
                    fx = _fix_from_x(y * _INV_SCALE)
                    plsc.store_scatter(buf, [offr, colv], fx, mask=m)

        load(wid, 0)
        load(wid + _NW, 1)

        @pl.loop(0, _TSTEPS // 2)
        def _pair(g):
            for b in range(2):
                t = 2 * g + b
                c = wid + _NW * t

                @pl.when((t >= 2) & (c - 2 * _NW < _NCHUNK))
                def _():
                    wait_store(c - 2 * _NW, b)

                @pl.when(c < _NCHUNK)
                def _():
                    wait_load(c, b)
                    process(c, b)
                    store(c, b)

                @pl.when(c + 2 * _NW < _NCHUNK)
                def _():
                    load(c + 2 * _NW, b)

    return k(xt, label)


def kernel(cosine, label):
    out_t = _sc_arc_margin_t(cosine.T, label.astype(jnp.int32))
    return out_t.T
